# baseline (speedup 1.0000x reference)
import functools

import jax
import jax.numpy as jnp
from jax import lax
from jax.experimental import pallas as pl
from jax.experimental.pallas import tpu as pltpu
from jax.experimental.pallas import tpu_sc as plsc

_NROWS = 8192
_NCOLS = 4096
_TC_BLOCK_ROWS = 512

_NUM_CORES = 2
_NUM_SUBCORES = 16
_NW = _NUM_CORES * _NUM_SUBCORES
_ROWS_PER_W = _NROWS // _NW
_SLAB_ROWS = 8
_SLABS_PER_W = _ROWS_PER_W // _SLAB_ROWS


def _argmax_body(x_ref, val_ref, col_ref):
    x = x_ref[...]
    m = jnp.max(x, axis=1, keepdims=True)
    col = lax.broadcasted_iota(jnp.int32, x.shape, 1)
    idx = jnp.min(jnp.where(x == m, col, jnp.int32(_NCOLS)), axis=1,
                  keepdims=True)
    val_ref[...] = m
    col_ref[...] = idx


def _tc_argmax(x):
    grid = (_NROWS // _TC_BLOCK_ROWS,)
    return pl.pallas_call(
        _argmax_body,
        grid=grid,
        in_specs=[pl.BlockSpec((_TC_BLOCK_ROWS, _NCOLS), lambda i: (i, 0))],
        out_specs=[
            pl.BlockSpec((_TC_BLOCK_ROWS, 1), lambda i: (i, 0)),
            pl.BlockSpec((_TC_BLOCK_ROWS, 1), lambda i: (i, 0)),
        ],
        out_shape=[
            jax.ShapeDtypeStruct((_NROWS, 1), jnp.float32),
            jax.ShapeDtypeStruct((_NROWS, 1), jnp.int32),
        ],
    )(x)


@functools.partial(
    pl.kernel,
    out_type=jax.ShapeDtypeStruct((_NROWS, _NCOLS), jnp.float32),
    mesh=plsc.VectorSubcoreMesh(core_axis_name="c", subcore_axis_name="s"),
    scratch_types=[
        pltpu.VMEM((_SLAB_ROWS, _NCOLS), jnp.float32),
        pltpu.VMEM((_SLAB_ROWS, _NCOLS), jnp.float32),
        pltpu.VMEM((_ROWS_PER_W + 16,), jnp.float32),
        pltpu.VMEM((_ROWS_PER_W + 16,), jnp.int32),
        pltpu.SemaphoreType.DMA,
        pltpu.SemaphoreType.DMA,
    ],
    compiler_params=pltpu.CompilerParams(use_tc_tiling_on_sc=True),
)
def _sc_zero_scatter(vals_hbm, cols_hbm, out_hbm, z0, z1, vv, cv, sem0, sem1):
    wid = lax.axis_index("s") * _NUM_CORES + lax.axis_index("c")
    pltpu.sync_copy(vals_hbm.at[pl.ds(wid * _ROWS_PER_W, _ROWS_PER_W)],
                    vv.at[pl.ds(0, _ROWS_PER_W)])
    pltpu.sync_copy(cols_hbm.at[pl.ds(wid * _ROWS_PER_W, _ROWS_PER_W)],
                    cv.at[pl.ds(0, _ROWS_PER_W)])

    zeros = jnp.zeros((16,), jnp.float32)
    lanes = lax.iota(jnp.int32, 16)

    def _zero_body(i, carry):
        for r in range(_SLAB_ROWS):
            z0[r, pl.ds(i * 16, 16)] = zeros
            z1[r, pl.ds(i * 16, 16)] = zeros
        return carry

    lax.fori_loop(0, _NCOLS // 16, _zero_body, 0)

    bufs = (z0, z1)
    sems = (sem0, sem1)
    handles = [None, None]
    prev = [None, None]
    row0 = wid * _ROWS_PER_W
    for s in range(_SLABS_PER_W):
        b = s % 2
        if handles[b] is not None:
            handles[b].wait()
            for r, b16 in prev[b]:
                bufs[b][r, pl.ds(b16, 16)] = zeros
        cw = cv[pl.ds(s * _SLAB_ROWS, 16)]
        vw = vv[pl.ds(s * _SLAB_ROWS, 16)]
        poked = []
        for r in range(_SLAB_ROWS):
            colr = cw[r]
            valr = vw[r]
            b16 = pl.multiple_of((colr >> 4) << 4, 16)
            lane = colr & 15
            w = bufs[b][r, pl.ds(b16, 16)]
            bufs[b][r, pl.ds(b16, 16)] = jnp.where(lanes == lane, valr, w)
            poked.append((r, b16))
        prev[b] = poked
        handles[b] = pltpu.async_copy(
            bufs[b],
            out_hbm.at[pl.ds(row0 + s * _SLAB_ROWS, _SLAB_ROWS), :],
            sems[b])
    handles[0].wait()
    handles[1].wait()


def kernel(x):
    vals2d, cols2d = _tc_argmax(x)
    return _sc_zero_scatter(vals2d.reshape(_NROWS), cols2d.reshape(_NROWS))


# speedup vs baseline: 1.0958x; 1.0958x over previous
import functools

import jax
import jax.numpy as jnp
from jax import lax
from jax.experimental import pallas as pl
from jax.experimental.pallas import tpu as pltpu
from jax.experimental.pallas import tpu_sc as plsc

_NROWS = 8192
_NCOLS = 4096
_TC_BLOCK_ROWS = 1024

_NUM_CORES = 2
_NUM_SUBCORES = 16
_NW = _NUM_CORES * _NUM_SUBCORES
_ROWS_PER_W = _NROWS // _NW
_SLAB_ROWS = 8
_SLABS_PER_W = _ROWS_PER_W // _SLAB_ROWS


def _argmax_body(x_ref, val_ref, col_ref):
    x = x_ref[...]
    m = jnp.max(x, axis=1, keepdims=True)
    col = lax.broadcasted_iota(jnp.int32, x.shape, 1)
    idx = jnp.min(jnp.where(x == m, col, jnp.int32(_NCOLS)), axis=1,
                  keepdims=True)
    val_ref[...] = m.reshape(val_ref.shape)
    col_ref[...] = idx.reshape(col_ref.shape)


_LANE_ROWS = _TC_BLOCK_ROWS // 128


def _tc_argmax(x):
    grid = (_NROWS // _TC_BLOCK_ROWS,)
    return pl.pallas_call(
        _argmax_body,
        grid=grid,
        in_specs=[pl.BlockSpec((_TC_BLOCK_ROWS, _NCOLS), lambda i: (i, 0))],
        out_specs=[
            pl.BlockSpec((_LANE_ROWS, 128), lambda i: (i, 0)),
            pl.BlockSpec((_LANE_ROWS, 128), lambda i: (i, 0)),
        ],
        out_shape=[
            jax.ShapeDtypeStruct((_NROWS // 128, 128), jnp.float32),
            jax.ShapeDtypeStruct((_NROWS // 128, 128), jnp.int32),
        ],
    )(x)


@functools.partial(
    pl.kernel,
    out_type=jax.ShapeDtypeStruct((_NROWS, _NCOLS), jnp.float32),
    mesh=plsc.VectorSubcoreMesh(core_axis_name="c", subcore_axis_name="s"),
    scratch_types=[
        pltpu.VMEM((_SLAB_ROWS, _NCOLS), jnp.float32),
        pltpu.VMEM((_SLAB_ROWS, _NCOLS), jnp.float32),
        pltpu.VMEM((_ROWS_PER_W + 16,), jnp.float32),
        pltpu.VMEM((_ROWS_PER_W + 16,), jnp.int32),
        pltpu.SemaphoreType.DMA,
        pltpu.SemaphoreType.DMA,
    ],
    compiler_params=pltpu.CompilerParams(use_tc_tiling_on_sc=True),
)
def _sc_zero_scatter(vals_hbm, cols_hbm, out_hbm, z0, z1, vv, cv, sem0, sem1):
    wid = lax.axis_index("s") * _NUM_CORES + lax.axis_index("c")
    pltpu.sync_copy(vals_hbm.at[pl.ds(wid * _ROWS_PER_W, _ROWS_PER_W)],
                    vv.at[pl.ds(0, _ROWS_PER_W)])
    pltpu.sync_copy(cols_hbm.at[pl.ds(wid * _ROWS_PER_W, _ROWS_PER_W)],
                    cv.at[pl.ds(0, _ROWS_PER_W)])

    zeros = jnp.zeros((16,), jnp.float32)
    lanes = lax.iota(jnp.int32, 16)

    def _zero_body(i, carry):
        for r in range(_SLAB_ROWS):
            z0[r, pl.ds(i * 16, 16)] = zeros
            z1[r, pl.ds(i * 16, 16)] = zeros
        return carry

    lax.fori_loop(0, _NCOLS // 16, _zero_body, 0)

    bufs = (z0, z1)
    sems = (sem0, sem1)
    handles = [None, None]
    prev = [None, None]
    row0 = wid * _ROWS_PER_W
    for s in range(_SLABS_PER_W):
        b = s % 2
        if handles[b] is not None:
            handles[b].wait()
            for r, b16 in prev[b]:
                bufs[b][r, pl.ds(b16, 16)] = zeros
        cw = cv[pl.ds(s * _SLAB_ROWS, 16)]
        vw = vv[pl.ds(s * _SLAB_ROWS, 16)]
        poked = []
        for r in range(_SLAB_ROWS):
            colr = cw[r]
            valr = vw[r]
            b16 = pl.multiple_of((colr >> 4) << 4, 16)
            lane = colr & 15
            w = bufs[b][r, pl.ds(b16, 16)]
            bufs[b][r, pl.ds(b16, 16)] = jnp.where(lanes == lane, valr, w)
            poked.append((r, b16))
        prev[b] = poked
        handles[b] = pltpu.async_copy(
            bufs[b],
            out_hbm.at[pl.ds(row0 + s * _SLAB_ROWS, _SLAB_ROWS), :],
            sems[b])
    handles[0].wait()
    handles[1].wait()


def kernel(x):
    vals2d, cols2d = _tc_argmax(x)
    return _sc_zero_scatter(vals2d.reshape(_NROWS), cols2d.reshape(_NROWS))


# speedup vs baseline: 1.1200x; 1.0221x over previous
import functools

import jax
import jax.numpy as jnp
from jax import lax
from jax.experimental import pallas as pl
from jax.experimental.pallas import tpu as pltpu
from jax.experimental.pallas import tpu_sc as plsc

_NROWS = 8192
_NCOLS = 4096
_TC_BLOCK_ROWS = 1024

_NUM_CORES = 2
_NUM_SUBCORES = 16
_NW = _NUM_CORES * _NUM_SUBCORES
_ROWS_PER_W = _NROWS // _NW
_SLAB_ROWS = 8
_SLABS_PER_W = _ROWS_PER_W // _SLAB_ROWS


def _argmax_body(x_ref, val_ref, col_ref):
    x = x_ref[...]
    m = jnp.max(x, axis=1, keepdims=True)
    col = lax.broadcasted_iota(jnp.int32, x.shape, 1)
    idx = jnp.min(jnp.where(x == m, col, jnp.int32(_NCOLS)), axis=1,
                  keepdims=True)
    val_ref[...] = m.reshape(val_ref.shape)
    col_ref[...] = idx.reshape(col_ref.shape)


_LANE_ROWS = _TC_BLOCK_ROWS // 128


def _tc_argmax(x):
    grid = (_NROWS // _TC_BLOCK_ROWS,)
    return pl.pallas_call(
        _argmax_body,
        grid=grid,
        in_specs=[pl.BlockSpec((_TC_BLOCK_ROWS, _NCOLS), lambda i: (i, 0))],
        out_specs=[
            pl.BlockSpec((_LANE_ROWS, 128), lambda i: (i, 0)),
            pl.BlockSpec((_LANE_ROWS, 128), lambda i: (i, 0)),
        ],
        out_shape=[
            jax.ShapeDtypeStruct((_NROWS // 128, 128), jnp.float32),
            jax.ShapeDtypeStruct((_NROWS // 128, 128), jnp.int32),
        ],
    )(x)


@functools.partial(
    pl.kernel,
    out_type=jax.ShapeDtypeStruct((_NROWS, _NCOLS), jnp.float32),
    mesh=plsc.VectorSubcoreMesh(core_axis_name="c", subcore_axis_name="s"),
    scratch_types=[
        pltpu.VMEM((_SLAB_ROWS, _NCOLS), jnp.float32),
        pltpu.VMEM((_SLAB_ROWS, _NCOLS), jnp.float32),
        pltpu.VMEM((_SLAB_ROWS, _NCOLS), jnp.float32),
        pltpu.VMEM((_ROWS_PER_W + 16,), jnp.float32),
        pltpu.VMEM((_ROWS_PER_W + 16,), jnp.int32),
        pltpu.SemaphoreType.DMA,
        pltpu.SemaphoreType.DMA,
        pltpu.SemaphoreType.DMA,
        pltpu.SemaphoreType.DMA,
    ],
    compiler_params=pltpu.CompilerParams(use_tc_tiling_on_sc=True),
)
def _sc_zero_scatter(vals_hbm, cols_hbm, out_hbm, z0, z1, z2, vv, cv,
                     sem0, sem1, sem2, stage_sem):
    wid = lax.axis_index("s") * _NUM_CORES + lax.axis_index("c")
    stage_v = pltpu.async_copy(
        vals_hbm.at[pl.ds(wid * _ROWS_PER_W, _ROWS_PER_W)],
        vv.at[pl.ds(0, _ROWS_PER_W)], stage_sem)
    stage_c = pltpu.async_copy(
        cols_hbm.at[pl.ds(wid * _ROWS_PER_W, _ROWS_PER_W)],
        cv.at[pl.ds(0, _ROWS_PER_W)], stage_sem)

    zeros = jnp.zeros((16,), jnp.float32)
    lanes = lax.iota(jnp.int32, 16)
    bufs = (z0, z1, z2)
    sems = (sem0, sem1, sem2)
    nb = len(bufs)

    def _zero_buf(z):
        def _zero_body(i, carry):
            for r in range(_SLAB_ROWS):
                z[r, pl.ds(i * 16, 16)] = zeros
            return carry
        lax.fori_loop(0, _NCOLS // 16, _zero_body, 0)

    row0 = wid * _ROWS_PER_W
    handles = [None] * nb
    prev = [None] * nb

    def _poke_and_fire(s):
        b = s % nb
        cw = cv[pl.ds(s * _SLAB_ROWS, 16)]
        vw = vv[pl.ds(s * _SLAB_ROWS, 16)]
        poked = []
        for r in range(_SLAB_ROWS):
            colr = cw[r]
            valr = vw[r]
            b16 = pl.multiple_of((colr >> 4) << 4, 16)
            lane = colr & 15
            w = bufs[b][r, pl.ds(b16, 16)]
            bufs[b][r, pl.ds(b16, 16)] = jnp.where(lanes == lane, valr, w)
            poked.append((r, b16))
        prev[b] = poked
        handles[b] = pltpu.async_copy(
            bufs[b],
            out_hbm.at[pl.ds(row0 + s * _SLAB_ROWS, _SLAB_ROWS), :],
            sems[b])

    _zero_buf(z0)
    stage_v.wait()
    stage_c.wait()
    _poke_and_fire(0)
    _zero_buf(z1)
    _poke_and_fire(1)
    _zero_buf(z2)
    _poke_and_fire(2)
    for s in range(nb, _SLABS_PER_W):
        b = s % nb
        handles[b].wait()
        for r, b16 in prev[b]:
            bufs[b][r, pl.ds(b16, 16)] = zeros
        _poke_and_fire(s)
    for h in handles:
        h.wait()


def kernel(x):
    vals2d, cols2d = _tc_argmax(x)
    return _sc_zero_scatter(vals2d.reshape(_NROWS), cols2d.reshape(_NROWS))
